# baseline (speedup 1.0000x reference)
import functools

import jax
import jax.numpy as jnp
from jax import lax
from jax.experimental import pallas as pl
from jax.experimental.pallas import tpu as pltpu
from jax.experimental.pallas import tpu_sc as plsc

NC = 2
NS = 16
L = 16

VOCAB = 128
EMBED = 2048
BATCH = 16384
SL = EMBED // 128

REP = 2
NW = NC * NS
BPW = BATCH // NW
K = 8
NCHUNK = BPW // K
NPAIR = NCHUNK // 2


def _body(pos_hbm, rowt_hbm, colt_hbm, out_hbm,
          pos_v, ridx_v, cidx_v,
          gr0, gc0, gr1, gc1, ob0, ob1,
          sr0, sc0, sr1, sc1, so0, so1):
    c = lax.axis_index("c")
    s = lax.axis_index("s")
    wid = s * NC + c
    base = wid * BPW

    for p in range(4):
        pltpu.sync_copy(pos_hbm.at[pl.ds(p * BATCH + base, BPW)],
                        pos_v.at[pl.ds(p * BPW, BPW)])

    def _quant(x):
        t = (x * float(VOCAB)).astype(jnp.int32)
        return jnp.minimum(t, VOCAB - 1)

    roff = lax.rem(wid, REP) * VOCAB

    def _idx_body(i, carry):
        r_lo = pos_v[pl.ds(0 * BPW + i * L, L)]
        c_lo = pos_v[pl.ds(1 * BPW + i * L, L)]
        r_hi = pos_v[pl.ds(2 * BPW + i * L, L)]
        c_hi = pos_v[pl.ds(3 * BPW + i * L, L)]
        ridx = lax.shift_right_logical(_quant(r_lo) + _quant(r_hi), 1)
        cidx = lax.shift_right_logical(_quant(c_lo) + _quant(c_hi), 1)
        ridx_v[pl.ds(i * L, L)] = ridx + roff
        cidx_v[pl.ds(i * L, L)] = cidx + roff
        return carry

    lax.fori_loop(0, BPW // L, _idx_body, 0)

    def _start_gather(cb, gr, gc, semr, semc):
        pltpu.async_copy(rowt_hbm.at[ridx_v.at[pl.ds(cb * K, K)]], gr, semr)
        pltpu.async_copy(colt_hbm.at[cidx_v.at[pl.ds(cb * K, K)]], gc, semc)

    def _wait_gather(cb, gr, gc, semr, semc):
        pltpu.make_async_copy(
            rowt_hbm.at[ridx_v.at[pl.ds(cb * K, K)]], gr, semr).wait()
        pltpu.make_async_copy(
            colt_hbm.at[cidx_v.at[pl.ds(cb * K, K)]], gc, semc).wait()

    def _start_out(cb, ob, semo):
        pltpu.async_copy(ob, out_hbm.at[pl.ds(base + cb * K, K)], semo)

    def _wait_out(ob, semo):
        pltpu.make_async_copy(ob, out_hbm.at[pl.ds(base, K)], semo).wait()

    def _add(gr, gc, ob):
        def _add_body(j, carry):
            for i in range(K):
                sl = (i, pl.ds(j * L, L))
                ob[sl] = gr[sl] + gc[sl]
            return carry
        lax.fori_loop(0, EMBED // L, _add_body, 0)

    _start_gather(0, gr0, gc0, sr0, sc0)

    def _pair_body(t, carry):
        a = 2 * t
        b = a + 1
        _start_gather(b, gr1, gc1, sr1, sc1)
        _wait_gather(a, gr0, gc0, sr0, sc0)

        @pl.when(t > 0)
        def _():
            _wait_out(ob0, so0)

        _add(gr0, gc0, ob0)
        _start_out(a, ob0, so0)

        @pl.when(t < NPAIR - 1)
        def _():
            _start_gather(a + 2, gr0, gc0, sr0, sc0)

        _wait_gather(b, gr1, gc1, sr1, sc1)

        @pl.when(t > 0)
        def _():
            _wait_out(ob1, so1)

        _add(gr1, gc1, ob1)
        _start_out(b, ob1, so1)
        return carry

    lax.fori_loop(0, NPAIR, _pair_body, 0)
    _wait_out(ob0, so0)
    _wait_out(ob1, so1)


@jax.jit
def _launch(pos_flat, row_embedding, col_embedding):
    mesh = plsc.VectorSubcoreMesh(core_axis_name="c", subcore_axis_name="s",
                                  num_cores=NC, num_subcores=NS)
    run = pl.kernel(
        _body,
        out_type=jax.ShapeDtypeStruct((BATCH, EMBED), jnp.float32),
        mesh=mesh,
        scratch_types=[
            pltpu.VMEM((BPW * 4,), jnp.float32),
            pltpu.VMEM((BPW,), jnp.int32),
            pltpu.VMEM((BPW,), jnp.int32),
            pltpu.VMEM((K, EMBED), jnp.float32),
            pltpu.VMEM((K, EMBED), jnp.float32),
            pltpu.VMEM((K, EMBED), jnp.float32),
            pltpu.VMEM((K, EMBED), jnp.float32),
            pltpu.VMEM((K, EMBED), jnp.float32),
            pltpu.VMEM((K, EMBED), jnp.float32),
            pltpu.SemaphoreType.DMA,
            pltpu.SemaphoreType.DMA,
            pltpu.SemaphoreType.DMA,
            pltpu.SemaphoreType.DMA,
            pltpu.SemaphoreType.DMA,
            pltpu.SemaphoreType.DMA,
        ],
    )
    return run(pos_flat, row_embedding, col_embedding)


def kernel(patch_pos, row_embedding, col_embedding, eval=1):
    del eval
    pos_flat = patch_pos.transpose(1, 2, 0).reshape(4 * BATCH)
    rep = lambda t: jnp.broadcast_to(
        t[None], (REP, VOCAB, EMBED)).reshape(REP * VOCAB, EMBED)
    return _launch(pos_flat, rep(row_embedding), rep(col_embedding))


# speedup vs baseline: 1.0280x; 1.0280x over previous
import functools

import jax
import jax.numpy as jnp
from jax import lax
from jax.experimental import pallas as pl
from jax.experimental.pallas import tpu as pltpu
from jax.experimental.pallas import tpu_sc as plsc

NC = 2
NS = 16
L = 16

VOCAB = 128
EMBED = 2048
BATCH = 16384
SL = EMBED // 128

REP = 4
NW = NC * NS
BPW = BATCH // NW
K = 8
NCHUNK = BPW // K
NPAIR = NCHUNK // 2


def _body(pos_hbm, rowt_hbm, colt_hbm, out_hbm,
          pos_v, ridx_v, cidx_v,
          gr0, gc0, gr1, gc1, ob0, ob1,
          sr0, sc0, sr1, sc1, so0, so1):
    c = lax.axis_index("c")
    s = lax.axis_index("s")
    wid = s * NC + c
    base = wid * BPW

    for p in range(4):
        pltpu.sync_copy(pos_hbm.at[pl.ds(p * BATCH + base, BPW)],
                        pos_v.at[pl.ds(p * BPW, BPW)])

    def _quant(x):
        t = (x * float(VOCAB)).astype(jnp.int32)
        return jnp.minimum(t, VOCAB - 1)

    roff = lax.rem(wid, REP) * VOCAB

    def _idx_body(i, carry):
        r_lo = pos_v[pl.ds(0 * BPW + i * L, L)]
        c_lo = pos_v[pl.ds(1 * BPW + i * L, L)]
        r_hi = pos_v[pl.ds(2 * BPW + i * L, L)]
        c_hi = pos_v[pl.ds(3 * BPW + i * L, L)]
        ridx = lax.shift_right_logical(_quant(r_lo) + _quant(r_hi), 1)
        cidx = lax.shift_right_logical(_quant(c_lo) + _quant(c_hi), 1)
        ridx_v[pl.ds(i * L, L)] = ridx + roff
        cidx_v[pl.ds(i * L, L)] = cidx + roff
        return carry

    lax.fori_loop(0, BPW // L, _idx_body, 0)

    def _start_gather(cb, gr, gc, semr, semc):
        pltpu.async_copy(rowt_hbm.at[ridx_v.at[pl.ds(cb * K, K)]], gr, semr)
        pltpu.async_copy(colt_hbm.at[cidx_v.at[pl.ds(cb * K, K)]], gc, semc)

    def _wait_gather(cb, gr, gc, semr, semc):
        pltpu.make_async_copy(
            rowt_hbm.at[ridx_v.at[pl.ds(cb * K, K)]], gr, semr).wait()
        pltpu.make_async_copy(
            colt_hbm.at[cidx_v.at[pl.ds(cb * K, K)]], gc, semc).wait()

    def _start_out(cb, ob, semo):
        pltpu.async_copy(ob, out_hbm.at[pl.ds(base + cb * K, K)], semo)

    def _wait_out(ob, semo):
        pltpu.make_async_copy(ob, out_hbm.at[pl.ds(base, K)], semo).wait()

    def _add(gr, gc, ob):
        def _add_body(j, carry):
            for i in range(K):
                sl = (i, pl.ds(j * L, L))
                ob[sl] = gr[sl] + gc[sl]
            return carry
        lax.fori_loop(0, EMBED // L, _add_body, 0)

    _start_gather(0, gr0, gc0, sr0, sc0)

    def _pair_body(t, carry):
        a = 2 * t
        b = a + 1
        _start_gather(b, gr1, gc1, sr1, sc1)
        _wait_gather(a, gr0, gc0, sr0, sc0)

        @pl.when(t > 0)
        def _():
            _wait_out(ob0, so0)

        _add(gr0, gc0, ob0)
        _start_out(a, ob0, so0)

        @pl.when(t < NPAIR - 1)
        def _():
            _start_gather(a + 2, gr0, gc0, sr0, sc0)

        _wait_gather(b, gr1, gc1, sr1, sc1)

        @pl.when(t > 0)
        def _():
            _wait_out(ob1, so1)

        _add(gr1, gc1, ob1)
        _start_out(b, ob1, so1)
        return carry

    lax.fori_loop(0, NPAIR, _pair_body, 0)
    _wait_out(ob0, so0)
    _wait_out(ob1, so1)


@jax.jit
def _launch(pos_flat, row_embedding, col_embedding):
    mesh = plsc.VectorSubcoreMesh(core_axis_name="c", subcore_axis_name="s",
                                  num_cores=NC, num_subcores=NS)
    run = pl.kernel(
        _body,
        out_type=jax.ShapeDtypeStruct((BATCH, EMBED), jnp.float32),
        mesh=mesh,
        compiler_params=pltpu.CompilerParams(needs_layout_passes=False),
        scratch_types=[
            pltpu.VMEM((BPW * 4,), jnp.float32),
            pltpu.VMEM((BPW,), jnp.int32),
            pltpu.VMEM((BPW,), jnp.int32),
            pltpu.VMEM((K, EMBED), jnp.float32),
            pltpu.VMEM((K, EMBED), jnp.float32),
            pltpu.VMEM((K, EMBED), jnp.float32),
            pltpu.VMEM((K, EMBED), jnp.float32),
            pltpu.VMEM((K, EMBED), jnp.float32),
            pltpu.VMEM((K, EMBED), jnp.float32),
            pltpu.SemaphoreType.DMA,
            pltpu.SemaphoreType.DMA,
            pltpu.SemaphoreType.DMA,
            pltpu.SemaphoreType.DMA,
            pltpu.SemaphoreType.DMA,
            pltpu.SemaphoreType.DMA,
        ],
    )
    return run(pos_flat, row_embedding, col_embedding)


def kernel(patch_pos, row_embedding, col_embedding, eval=1):
    del eval
    pos_flat = patch_pos.transpose(1, 2, 0).reshape(4 * BATCH)
    rep = lambda t: jnp.broadcast_to(
        t[None], (REP, VOCAB, EMBED)).reshape(REP * VOCAB, EMBED)
    return _launch(pos_flat, rep(row_embedding), rep(col_embedding))
